# baseline (speedup 1.0000x reference)
import functools

import jax
import jax.numpy as jnp
from jax import lax
from jax.experimental import pallas as pl
from jax.experimental.pallas import tpu as pltpu
from jax.experimental.pallas import tpu_sc as plsc

N = 2048
D = 768
H = 768
E = 64
T = 128
TT = 32
G = 95
NP = 4096
NW = 32
TOK_W = N // NW
NEG_INF = float("-inf")


def _router_body(x_ref, wr_ref, br_ref, pos_ref, sc_ref, te_ref,
                 trb_ref, rs_ref, re_ref, nt_ref):
    xf = x_ref[...]
    logits = jnp.dot(xf, wr_ref[...], preferred_element_type=jnp.float32)
    logits = logits + br_ref[...]
    col = lax.broadcasted_iota(jnp.int32, (N, E), 1)

    m0 = jnp.max(logits, axis=1, keepdims=True)
    a0 = jnp.min(jnp.where(logits == m0, col, E), axis=1)
    oh0 = (col == a0[:, None]).astype(jnp.float32)
    neg = jnp.where(col == a0[:, None], NEG_INF, logits)
    m1 = jnp.max(neg, axis=1, keepdims=True)
    a1 = jnp.min(jnp.where(neg == m1, col, E), axis=1)
    oh1 = (col == a1[:, None]).astype(jnp.float32)

    t = jnp.exp(m1 - m0)
    w0 = 1.0 / (1.0 + t)
    w1 = t * w0

    cnt = jnp.sum(oh0, axis=0, keepdims=True) + jnp.sum(oh1, axis=0, keepdims=True)
    ecol = lax.broadcasted_iota(jnp.int32, (E, E), 1)
    erow = lax.broadcasted_iota(jnp.int32, (E, E), 0)
    ls_strict = (ecol < erow).astype(jnp.float32)
    off = jnp.dot(ls_strict, cnt.reshape(E, 1),
                  preferred_element_type=jnp.float32).reshape(1, E)

    seg_e = off + cnt
    tstart = (lax.broadcasted_iota(jnp.int32, (TT, E), 0) * T).astype(jnp.float32)
    os_ = jnp.maximum(off, tstart)
    oe_ = jnp.minimum(seg_e, tstart + T)
    validm = (oe_ > os_).astype(jnp.float32)
    m_est = (erow < ecol).astype(jnp.float32)
    within = jnp.dot(validm, m_est, preferred_element_type=jnp.float32)
    rowtot = jnp.sum(validm, axis=1, keepdims=True)
    trow = lax.broadcasted_iota(jnp.int32, (TT, TT), 0)
    tcol = lax.broadcasted_iota(jnp.int32, (TT, TT), 1)
    lt_strict = (tcol < trow).astype(jnp.float32)
    rowexcl = jnp.dot(lt_strict, rowtot, preferred_element_type=jnp.float32)
    rankp = within + rowexcl
    ntot = jnp.sum(validm)

    g3 = lax.broadcasted_iota(jnp.int32, (TT, E, 128), 2).astype(jnp.float32)
    g3 = jnp.minimum(g3, ntot - 1.0)
    sel = jnp.where((rankp[:, :, None] == g3) & (validm[:, :, None] > 0.0),
                    1.0, 0.0)
    e3 = lax.broadcasted_iota(jnp.int32, (TT, E, 128), 1).astype(jnp.float32)
    t3 = lax.broadcasted_iota(jnp.int32, (TT, E, 128), 0).astype(jnp.float32)
    te_g = jnp.sum(sel * e3, axis=(0, 1))
    trb_g = jnp.sum(sel * t3, axis=(0, 1))
    rs_g = jnp.sum(sel * (os_ - tstart)[:, :, None], axis=(0, 1))
    re_g = jnp.sum(sel * (oe_ - tstart)[:, :, None], axis=(0, 1))
    te_ref[...] = jnp.broadcast_to(te_g.reshape(1, 128).astype(jnp.int32), (8, 128))
    trb_ref[...] = jnp.broadcast_to(trb_g.reshape(1, 128).astype(jnp.int32), (8, 128))
    rs_ref[...] = jnp.broadcast_to(rs_g.reshape(1, 128).astype(jnp.int32), (8, 128))
    re_ref[...] = jnp.broadcast_to(re_g.reshape(1, 128).astype(jnp.int32), (8, 128))
    nt_ref[...] = jnp.broadcast_to(ntot.astype(jnp.int32), (8, 128))

    nrow = lax.broadcasted_iota(jnp.int32, (N, N), 0)
    ncol = lax.broadcasted_iota(jnp.int32, (N, N), 1)
    ln_strict = (ncol < nrow).astype(jnp.bfloat16)
    oh01 = jnp.concatenate([oh0, oh1], axis=1).astype(jnp.bfloat16)
    r01 = jnp.dot(ln_strict, oh01, preferred_element_type=jnp.float32)
    tot0 = jnp.sum(oh0, axis=0, keepdims=True)

    rank0 = jnp.sum(r01[:, :E] * oh0, axis=1)
    rank1 = jnp.sum((r01[:, E:] + tot0) * oh1, axis=1)
    base0 = jnp.sum(oh0 * off, axis=1)
    base1 = jnp.sum(oh1 * off, axis=1)
    pos0 = (base0 + rank0).astype(jnp.int32)
    pos1 = (base1 + rank1).astype(jnp.int32)

    rowsel = lax.broadcasted_iota(jnp.int32, (8, N), 0)
    pos_ref[...] = jnp.where(rowsel == 0, pos0[None, :],
                             jnp.where(rowsel == 1, pos1[None, :], 0))
    sc_ref[...] = jnp.where(rowsel == 0, w0[:, 0][None, :],
                            jnp.where(rowsel == 1, w1[:, 0][None, :], 0.0))


def _router(x2d, Wr, br2d):
    return pl.pallas_call(
        _router_body,
        out_shape=(
            jax.ShapeDtypeStruct((8, N), jnp.int32),
            jax.ShapeDtypeStruct((8, N), jnp.float32),
            jax.ShapeDtypeStruct((8, 128), jnp.int32),
            jax.ShapeDtypeStruct((8, 128), jnp.int32),
            jax.ShapeDtypeStruct((8, 128), jnp.int32),
            jax.ShapeDtypeStruct((8, 128), jnp.int32),
            jax.ShapeDtypeStruct((8, 128), jnp.int32),
        ),
    )(x2d, Wr, br2d)


def _gmm_body(te_ref, trb_ref, rs_ref, re_ref, nt_ref, xg_ref, w1_ref, b1_ref,
              w2_ref, b2_ref, y_ref):
    del te_ref, trb_ref
    g = pl.program_id(0)

    @pl.when(g < nt_ref[0])
    def _():
        xb = xg_ref[...].astype(jnp.bfloat16)
        h = jnp.dot(xb, w1_ref[0].astype(jnp.bfloat16),
                    preferred_element_type=jnp.float32) + b1_ref[0]
        h = jnp.maximum(h, 0.0).astype(jnp.bfloat16)
        y = jnp.dot(h, w2_ref[0].astype(jnp.bfloat16),
                    preferred_element_type=jnp.float32) + b2_ref[0]
        riota = lax.broadcasted_iota(jnp.int32, (T, 1), 0)
        mask = (riota >= rs_ref[g]) & (riota < re_ref[g])
        y_ref[...] = jnp.where(mask, y, y_ref[...])


def _gmm(te, trb, rs, re, nt, xg, W1, b1, W2, b2):
    grid_spec = pltpu.PrefetchScalarGridSpec(
        num_scalar_prefetch=5,
        grid=(G,),
        in_specs=[
            pl.BlockSpec((T, D), lambda g, te, trb, rs, re, nt: (trb[g], 0)),
            pl.BlockSpec((1, D, H), lambda g, te, trb, rs, re, nt: (te[g], 0, 0)),
            pl.BlockSpec((1, 1, H), lambda g, te, trb, rs, re, nt: (te[g], 0, 0)),
            pl.BlockSpec((1, H, D), lambda g, te, trb, rs, re, nt: (te[g], 0, 0)),
            pl.BlockSpec((1, 1, D), lambda g, te, trb, rs, re, nt: (te[g], 0, 0)),
        ],
        out_specs=pl.BlockSpec((T, D), lambda g, te, trb, rs, re, nt: (trb[g], 0)),
    )
    return pl.pallas_call(
        _gmm_body,
        grid_spec=grid_spec,
        out_shape=jax.ShapeDtypeStruct((NP, D), jnp.float32),
        compiler_params=pltpu.CompilerParams(
            dimension_semantics=("arbitrary",)),
    )(te, trb, rs, re, nt, xg, W1, b1.reshape(E, 1, H), W2, b2.reshape(E, 1, D))


def _dispatch_body(x_hbm, pos_hbm, xg_hbm, xloc, p0, p1, sem):
    wid = lax.axis_index("s") * 2 + lax.axis_index("c")
    t0 = wid * TOK_W
    pltpu.sync_copy(x_hbm.at[pl.ds(t0, TOK_W)], xloc)
    pltpu.sync_copy(pos_hbm.at[0, pl.ds(t0, TOK_W)], p0)
    pltpu.sync_copy(pos_hbm.at[1, pl.ds(t0, TOK_W)], p1)
    pltpu.async_copy(xloc, xg_hbm.at[p0], sem).wait()
    pltpu.async_copy(xloc, xg_hbm.at[p1], sem).wait()


def _dispatch(x2d, pos):
    mesh = plsc.VectorSubcoreMesh(core_axis_name="c", subcore_axis_name="s")
    f = pl.kernel(
        _dispatch_body,
        out_type=jax.ShapeDtypeStruct((NP, D), jnp.float32),
        mesh=mesh,
        scratch_types=[
            pltpu.VMEM((TOK_W, D), jnp.float32),
            pltpu.VMEM((TOK_W,), jnp.int32),
            pltpu.VMEM((TOK_W,), jnp.int32),
            pltpu.SemaphoreType.DMA,
        ],
    )
    return f(x2d, pos)


def _combine_body(ys_hbm, pos_hbm, sc_hbm, out_hbm, y0, y1, p0, p1, s0, s1, sem):
    wid = lax.axis_index("s") * 2 + lax.axis_index("c")
    t0 = wid * TOK_W
    pltpu.sync_copy(pos_hbm.at[0, pl.ds(t0, TOK_W)], p0)
    pltpu.sync_copy(pos_hbm.at[1, pl.ds(t0, TOK_W)], p1)
    pltpu.sync_copy(sc_hbm.at[0, pl.ds(t0, TOK_W)], s0)
    pltpu.sync_copy(sc_hbm.at[1, pl.ds(t0, TOK_W)], s1)
    pltpu.async_copy(ys_hbm.at[p0], y0, sem).wait()
    pltpu.async_copy(ys_hbm.at[p1], y1, sem).wait()

    @plsc.parallel_loop(0, TOK_W, unroll=2)
    def _(r):
        ridx = jnp.zeros((16,), jnp.int32) + r
        w0 = plsc.load_gather(s0, [ridx])
        w1 = plsc.load_gather(s1, [ridx])
        for c in range(D // 16):
            sl = pl.ds(16 * c, 16)
            y0[r, sl] = y0[r, sl] * w0 + y1[r, sl] * w1
    pltpu.sync_copy(y0, out_hbm.at[pl.ds(t0, TOK_W)])


def _combine(ys, pos, sc):
    mesh = plsc.VectorSubcoreMesh(core_axis_name="c", subcore_axis_name="s")
    f = pl.kernel(
        _combine_body,
        out_type=jax.ShapeDtypeStruct((N, D), jnp.float32),
        mesh=mesh,
        scratch_types=[
            pltpu.VMEM((TOK_W, D), jnp.float32),
            pltpu.VMEM((TOK_W, D), jnp.float32),
            pltpu.VMEM((TOK_W,), jnp.int32),
            pltpu.VMEM((TOK_W,), jnp.int32),
            pltpu.VMEM((TOK_W,), jnp.float32),
            pltpu.VMEM((TOK_W,), jnp.float32),
            pltpu.SemaphoreType.DMA,
        ],
        compiler_params=pltpu.CompilerParams(needs_layout_passes=False),
    )
    return f(ys, pos, sc)


def kernel(x, Wr, br, W1, b1, W2, b2):
    x2d = x.reshape(N, D)
    pos, sc, te8, trb8, rs8, re8, nt8 = _router(x2d, Wr, br.reshape(1, E))
    te = te8[0, :G]
    trb = trb8[0, :G]
    rs = rs8[0, :G]
    re = re8[0, :G]
    nt = nt8[0, :1]
    xg = _dispatch(x2d, pos)
    ys = _gmm(te, trb, rs, re, nt, xg, W1, b1, W2, b2)
    out = _combine(ys, pos, sc)
    return out.reshape(1, N, D)


# speedup vs baseline: 1.1937x; 1.1937x over previous
import functools

import jax
import jax.numpy as jnp
from jax import lax
from jax.experimental import pallas as pl
from jax.experimental.pallas import tpu as pltpu
from jax.experimental.pallas import tpu_sc as plsc

N = 2048
D = 768
H = 768
E = 64
T = 128
G = 95
NP = 12288
NW = 32
TOK_W = N // NW
NEG_INF = float("-inf")


def _router_body(x_ref, wr_ref, br_ref, pos_ref, sc_ref, te_ref,
                 trb_ref, nt_ref):
    xf = x_ref[...]
    logits = jnp.dot(xf, wr_ref[...], preferred_element_type=jnp.float32)
    logits = logits + br_ref[...]
    col = lax.broadcasted_iota(jnp.int32, (N, E), 1)

    m0 = jnp.max(logits, axis=1, keepdims=True)
    a0 = jnp.min(jnp.where(logits == m0, col, E), axis=1)
    oh0 = (col == a0[:, None]).astype(jnp.float32)
    neg = jnp.where(col == a0[:, None], NEG_INF, logits)
    m1 = jnp.max(neg, axis=1, keepdims=True)
    a1 = jnp.min(jnp.where(neg == m1, col, E), axis=1)
    oh1 = (col == a1[:, None]).astype(jnp.float32)

    t = jnp.exp(m1 - m0)
    w0 = 1.0 / (1.0 + t)
    w1 = t * w0

    cnt = jnp.sum(oh0, axis=0, keepdims=True) + jnp.sum(oh1, axis=0, keepdims=True)
    ecol = lax.broadcasted_iota(jnp.int32, (E, E), 1)
    erow = lax.broadcasted_iota(jnp.int32, (E, E), 0)
    ls_strict = (ecol < erow).astype(jnp.float32)
    ls_incl = (ecol <= erow).astype(jnp.float32)
    ntiles = jnp.floor((cnt + (T - 1)) * (1.0 / T))
    pc = ntiles * T
    off = jnp.dot(ls_strict, pc.reshape(E, 1),
                  preferred_element_type=jnp.float32).reshape(1, E)
    c_incl = jnp.dot(ls_incl, ntiles.reshape(E, 1),
                     preferred_element_type=jnp.float32).reshape(1, E)
    tse = c_incl - ntiles
    total_tiles = jnp.max(c_incl)

    gg = lax.broadcasted_iota(jnp.int32, (128, E), 0).astype(jnp.float32)
    gclamp = jnp.minimum(gg, total_tiles - 1.0)
    e_g = jnp.sum((c_incl <= gclamp).astype(jnp.float32), axis=1, keepdims=True)
    gcol = lax.broadcasted_iota(jnp.int32, (128, E), 1).astype(jnp.float32)
    ohg = (gcol == e_g).astype(jnp.float32)
    off_g = jnp.sum(ohg * off, axis=1, keepdims=True)
    tse_g = jnp.sum(ohg * tse, axis=1, keepdims=True)
    rb_g = off_g * (1.0 / T) + (gclamp[:, 0:1] - tse_g)
    te_ref[...] = jnp.broadcast_to(e_g.reshape(1, 128).astype(jnp.int32), (8, 128))
    trb_ref[...] = jnp.broadcast_to(rb_g.reshape(1, 128).astype(jnp.int32), (8, 128))
    nt_ref[...] = jnp.broadcast_to(total_tiles.astype(jnp.int32), (8, 128))

    nrow = lax.broadcasted_iota(jnp.int32, (N, N), 0)
    ncol = lax.broadcasted_iota(jnp.int32, (N, N), 1)
    ln_strict = (ncol < nrow).astype(jnp.bfloat16)
    oh01 = jnp.concatenate([oh0, oh1], axis=1).astype(jnp.bfloat16)
    r01 = jnp.dot(ln_strict, oh01, preferred_element_type=jnp.float32)
    tot0 = jnp.sum(oh0, axis=0, keepdims=True)

    rank0 = jnp.sum(r01[:, :E] * oh0, axis=1)
    rank1 = jnp.sum((r01[:, E:] + tot0) * oh1, axis=1)
    base0 = jnp.sum(oh0 * off, axis=1)
    base1 = jnp.sum(oh1 * off, axis=1)
    pos0 = (base0 + rank0).astype(jnp.int32)
    pos1 = (base1 + rank1).astype(jnp.int32)

    rowsel = lax.broadcasted_iota(jnp.int32, (8, N), 0)
    pos_ref[...] = jnp.where(rowsel == 0, pos0[None, :],
                             jnp.where(rowsel == 1, pos1[None, :], 0))
    sc_ref[...] = jnp.where(rowsel == 0, w0[:, 0][None, :],
                            jnp.where(rowsel == 1, w1[:, 0][None, :], 0.0))


def _router(x2d, Wr, br2d):
    return pl.pallas_call(
        _router_body,
        out_shape=(
            jax.ShapeDtypeStruct((8, N), jnp.int32),
            jax.ShapeDtypeStruct((8, N), jnp.float32),
            jax.ShapeDtypeStruct((8, 128), jnp.int32),
            jax.ShapeDtypeStruct((8, 128), jnp.int32),
            jax.ShapeDtypeStruct((8, 128), jnp.int32),
        ),
    )(x2d, Wr, br2d)


def _gmm_body(te_ref, trb_ref, nt_ref, xg_ref, w1_ref, b1_ref, w2_ref, b2_ref,
              y_ref):
    del te_ref, trb_ref

    @pl.when(pl.program_id(0) < nt_ref[0])
    def _():
        xb = xg_ref[...].astype(jnp.bfloat16)
        h = jnp.dot(xb, w1_ref[0].astype(jnp.bfloat16),
                    preferred_element_type=jnp.float32) + b1_ref[0]
        h = jnp.maximum(h, 0.0).astype(jnp.bfloat16)
        y = jnp.dot(h, w2_ref[0].astype(jnp.bfloat16),
                    preferred_element_type=jnp.float32) + b2_ref[0]
        y_ref[...] = y


def _gmm(te, trb, nt, xg, W1, b1, W2, b2):
    grid_spec = pltpu.PrefetchScalarGridSpec(
        num_scalar_prefetch=3,
        grid=(G,),
        in_specs=[
            pl.BlockSpec((T, D), lambda g, te, trb, nt: (trb[g], 0)),
            pl.BlockSpec((1, D, H), lambda g, te, trb, nt: (te[g], 0, 0)),
            pl.BlockSpec((1, 1, H), lambda g, te, trb, nt: (te[g], 0, 0)),
            pl.BlockSpec((1, H, D), lambda g, te, trb, nt: (te[g], 0, 0)),
            pl.BlockSpec((1, 1, D), lambda g, te, trb, nt: (te[g], 0, 0)),
        ],
        out_specs=pl.BlockSpec((T, D), lambda g, te, trb, nt: (trb[g], 0)),
    )
    return pl.pallas_call(
        _gmm_body,
        grid_spec=grid_spec,
        out_shape=jax.ShapeDtypeStruct((NP, D), jnp.float32),
        compiler_params=pltpu.CompilerParams(
            dimension_semantics=("arbitrary",)),
    )(te, trb, nt, xg, W1, b1.reshape(E, 1, H), W2, b2.reshape(E, 1, D))


def _dispatch_body(x_hbm, pos_hbm, xg_hbm, xloc, p0, p1, sem):
    wid = lax.axis_index("s") * 2 + lax.axis_index("c")
    t0 = wid * TOK_W
    pltpu.sync_copy(x_hbm.at[pl.ds(t0, TOK_W)], xloc)
    pltpu.sync_copy(pos_hbm.at[0, pl.ds(t0, TOK_W)], p0)
    pltpu.sync_copy(pos_hbm.at[1, pl.ds(t0, TOK_W)], p1)
    pltpu.async_copy(xloc, xg_hbm.at[p0], sem).wait()
    pltpu.async_copy(xloc, xg_hbm.at[p1], sem).wait()


def _dispatch(x2d, pos):
    mesh = plsc.VectorSubcoreMesh(core_axis_name="c", subcore_axis_name="s")
    f = pl.kernel(
        _dispatch_body,
        out_type=jax.ShapeDtypeStruct((NP, D), jnp.float32),
        mesh=mesh,
        scratch_types=[
            pltpu.VMEM((TOK_W, D), jnp.float32),
            pltpu.VMEM((TOK_W,), jnp.int32),
            pltpu.VMEM((TOK_W,), jnp.int32),
            pltpu.SemaphoreType.DMA,
        ],
    )
    return f(x2d, pos)


def _combine_body(ys_hbm, pos_hbm, sc_hbm, out_hbm, y0, y1, p0, p1, s0, s1, sem):
    wid = lax.axis_index("s") * 2 + lax.axis_index("c")
    t0 = wid * TOK_W
    pltpu.sync_copy(pos_hbm.at[0, pl.ds(t0, TOK_W)], p0)
    pltpu.sync_copy(pos_hbm.at[1, pl.ds(t0, TOK_W)], p1)
    pltpu.sync_copy(sc_hbm.at[0, pl.ds(t0, TOK_W)], s0)
    pltpu.sync_copy(sc_hbm.at[1, pl.ds(t0, TOK_W)], s1)
    pltpu.async_copy(ys_hbm.at[p0], y0, sem).wait()
    pltpu.async_copy(ys_hbm.at[p1], y1, sem).wait()

    @plsc.parallel_loop(0, TOK_W, unroll=2)
    def _(r):
        ridx = jnp.zeros((16,), jnp.int32) + r
        w0 = plsc.load_gather(s0, [ridx])
        w1 = plsc.load_gather(s1, [ridx])
        for c in range(D // 16):
            sl = pl.ds(16 * c, 16)
            y0[r, sl] = y0[r, sl] * w0 + y1[r, sl] * w1
    pltpu.sync_copy(y0, out_hbm.at[pl.ds(t0, TOK_W)])


def _combine(ys, pos, sc):
    mesh = plsc.VectorSubcoreMesh(core_axis_name="c", subcore_axis_name="s")
    f = pl.kernel(
        _combine_body,
        out_type=jax.ShapeDtypeStruct((N, D), jnp.float32),
        mesh=mesh,
        scratch_types=[
            pltpu.VMEM((TOK_W, D), jnp.float32),
            pltpu.VMEM((TOK_W, D), jnp.float32),
            pltpu.VMEM((TOK_W,), jnp.int32),
            pltpu.VMEM((TOK_W,), jnp.int32),
            pltpu.VMEM((TOK_W,), jnp.float32),
            pltpu.VMEM((TOK_W,), jnp.float32),
            pltpu.SemaphoreType.DMA,
        ],
        compiler_params=pltpu.CompilerParams(needs_layout_passes=False),
    )
    return f(ys, pos, sc)


def kernel(x, Wr, br, W1, b1, W2, b2):
    x2d = x.reshape(N, D)
    pos, sc, te8, trb8, nt8 = _router(x2d, Wr, br.reshape(1, E))
    te = te8[0, :G]
    trb = trb8[0, :G]
    nt = nt8[0, :1]
    xg = _dispatch(x2d, pos)
    ys = _gmm(te, trb, nt, xg, W1, b1, W2, b2)
    out = _combine(ys, pos, sc)
    return out.reshape(1, N, D)


# speedup vs baseline: 1.1976x; 1.0033x over previous
import functools

import jax
import jax.numpy as jnp
from jax import lax
from jax.experimental import pallas as pl
from jax.experimental.pallas import tpu as pltpu
from jax.experimental.pallas import tpu_sc as plsc

N = 2048
D = 768
H = 768
E = 64
T = 128
G = 95
NP = 12288
NW = 32
TOK_W = N // NW
NEG_INF = float("-inf")


def _router_body(x_ref, wr_ref, br_ref, pos_ref, sc_ref, te_ref,
                 trb_ref, nt_ref):
    xf = x_ref[...]
    logits = jnp.dot(xf, wr_ref[...], preferred_element_type=jnp.float32)
    logits = logits + br_ref[...]
    col = lax.broadcasted_iota(jnp.int32, (N, E), 1)

    m0 = jnp.max(logits, axis=1, keepdims=True)
    a0 = jnp.min(jnp.where(logits == m0, col, E), axis=1)
    oh0 = (col == a0[:, None]).astype(jnp.float32)
    neg = jnp.where(col == a0[:, None], NEG_INF, logits)
    m1 = jnp.max(neg, axis=1, keepdims=True)
    a1 = jnp.min(jnp.where(neg == m1, col, E), axis=1)
    oh1 = (col == a1[:, None]).astype(jnp.float32)

    t = jnp.exp(m1 - m0)
    w0 = 1.0 / (1.0 + t)
    w1 = t * w0

    cnt = jnp.sum(oh0, axis=0, keepdims=True) + jnp.sum(oh1, axis=0, keepdims=True)
    ecol = lax.broadcasted_iota(jnp.int32, (E, E), 1)
    erow = lax.broadcasted_iota(jnp.int32, (E, E), 0)
    ls_strict = (ecol < erow).astype(jnp.float32)
    ls_incl = (ecol <= erow).astype(jnp.float32)
    ntiles = jnp.floor((cnt + (T - 1)) * (1.0 / T))
    pc = ntiles * T
    off = jnp.dot(ls_strict, pc.reshape(E, 1),
                  preferred_element_type=jnp.float32).reshape(1, E)
    c_incl = jnp.dot(ls_incl, ntiles.reshape(E, 1),
                     preferred_element_type=jnp.float32).reshape(1, E)
    tse = c_incl - ntiles
    total_tiles = jnp.max(c_incl)

    gg = lax.broadcasted_iota(jnp.int32, (128, E), 0).astype(jnp.float32)
    gclamp = jnp.minimum(gg, total_tiles - 1.0)
    e_g = jnp.sum((c_incl <= gclamp).astype(jnp.float32), axis=1, keepdims=True)
    gcol = lax.broadcasted_iota(jnp.int32, (128, E), 1).astype(jnp.float32)
    ohg = (gcol == e_g).astype(jnp.float32)
    off_g = jnp.sum(ohg * off, axis=1, keepdims=True)
    tse_g = jnp.sum(ohg * tse, axis=1, keepdims=True)
    rb_g = off_g * (1.0 / T) + (gclamp[:, 0:1] - tse_g)
    te_ref[...] = jnp.broadcast_to(e_g.reshape(1, 128).astype(jnp.int32), (8, 128))
    trb_ref[...] = jnp.broadcast_to(rb_g.reshape(1, 128).astype(jnp.int32), (8, 128))
    nt_ref[...] = jnp.broadcast_to(total_tiles.astype(jnp.int32), (8, 128))

    nrow = lax.broadcasted_iota(jnp.int32, (N, N), 0)
    ncol = lax.broadcasted_iota(jnp.int32, (N, N), 1)
    ln_strict = (ncol < nrow).astype(jnp.bfloat16)
    oh01 = jnp.concatenate([oh0, oh1], axis=1).astype(jnp.bfloat16)
    r01 = jnp.dot(ln_strict, oh01, preferred_element_type=jnp.float32)
    tot0 = jnp.sum(oh0, axis=0, keepdims=True)

    rank0 = jnp.sum(r01[:, :E] * oh0, axis=1)
    rank1 = jnp.sum((r01[:, E:] + tot0) * oh1, axis=1)
    base0 = jnp.sum(oh0 * off, axis=1)
    base1 = jnp.sum(oh1 * off, axis=1)
    pos0 = (base0 + rank0).astype(jnp.int32)
    pos1 = (base1 + rank1).astype(jnp.int32)

    rowsel = lax.broadcasted_iota(jnp.int32, (8, N), 0)
    pos_ref[...] = jnp.where(rowsel == 0, pos0[None, :],
                             jnp.where(rowsel == 1, pos1[None, :], 0))
    sc_ref[...] = jnp.where(rowsel == 0, w0[:, 0][None, :],
                            jnp.where(rowsel == 1, w1[:, 0][None, :], 0.0))


def _router(x2d, Wr, br2d):
    return pl.pallas_call(
        _router_body,
        out_shape=(
            jax.ShapeDtypeStruct((8, N), jnp.int32),
            jax.ShapeDtypeStruct((8, N), jnp.float32),
            jax.ShapeDtypeStruct((8, 128), jnp.int32),
            jax.ShapeDtypeStruct((8, 128), jnp.int32),
            jax.ShapeDtypeStruct((8, 128), jnp.int32),
        ),
    )(x2d, Wr, br2d)


def _gmm_body(te_ref, trb_ref, nt_ref, xg_ref, w1_ref, b1_ref, w2_ref, b2_ref,
              y_ref):
    del te_ref, trb_ref

    @pl.when(pl.program_id(0) < nt_ref[0])
    def _():
        xb = xg_ref[...].astype(jnp.bfloat16)
        h = jnp.dot(xb, w1_ref[0].astype(jnp.bfloat16),
                    preferred_element_type=jnp.float32) + b1_ref[0]
        h = jnp.maximum(h, 0.0).astype(jnp.bfloat16)
        y = jnp.dot(h, w2_ref[0].astype(jnp.bfloat16),
                    preferred_element_type=jnp.float32) + b2_ref[0]
        y_ref[...] = y


def _gmm(te, trb, nt, xg, W1, b1, W2, b2):
    grid_spec = pltpu.PrefetchScalarGridSpec(
        num_scalar_prefetch=3,
        grid=(G,),
        in_specs=[
            pl.BlockSpec((T, D), lambda g, te, trb, nt: (trb[g], 0)),
            pl.BlockSpec((1, D, H), lambda g, te, trb, nt: (te[g], 0, 0)),
            pl.BlockSpec((1, 1, H), lambda g, te, trb, nt: (te[g], 0, 0)),
            pl.BlockSpec((1, H, D), lambda g, te, trb, nt: (te[g], 0, 0)),
            pl.BlockSpec((1, 1, D), lambda g, te, trb, nt: (te[g], 0, 0)),
        ],
        out_specs=pl.BlockSpec((T, D), lambda g, te, trb, nt: (trb[g], 0)),
    )
    return pl.pallas_call(
        _gmm_body,
        grid_spec=grid_spec,
        out_shape=jax.ShapeDtypeStruct((NP, D), jnp.float32),
        compiler_params=pltpu.CompilerParams(
            dimension_semantics=("arbitrary",)),
    )(te, trb, nt, xg, W1, b1.reshape(E, 1, H), W2, b2.reshape(E, 1, D))


def _dispatch_body(x_hbm, pos_hbm, xg_hbm, xloc, p0, p1, sem):
    wid = lax.axis_index("s") * 2 + lax.axis_index("c")
    t0 = wid * TOK_W
    pltpu.sync_copy(x_hbm.at[pl.ds(t0, TOK_W)], xloc)
    pltpu.sync_copy(pos_hbm.at[0, pl.ds(t0, TOK_W)], p0)
    pltpu.sync_copy(pos_hbm.at[1, pl.ds(t0, TOK_W)], p1)
    cp0 = pltpu.async_copy(xloc, xg_hbm.at[p0], sem)
    cp1 = pltpu.async_copy(xloc, xg_hbm.at[p1], sem)
    cp0.wait()
    cp1.wait()


def _dispatch(x2d, pos):
    mesh = plsc.VectorSubcoreMesh(core_axis_name="c", subcore_axis_name="s")
    f = pl.kernel(
        _dispatch_body,
        out_type=jax.ShapeDtypeStruct((NP, D), jnp.float32),
        mesh=mesh,
        scratch_types=[
            pltpu.VMEM((TOK_W, D), jnp.float32),
            pltpu.VMEM((TOK_W,), jnp.int32),
            pltpu.VMEM((TOK_W,), jnp.int32),
            pltpu.SemaphoreType.DMA,
        ],
    )
    return f(x2d, pos)


def _combine_body(ys_hbm, pos_hbm, sc_hbm, out_hbm, y0, y1, p0, p1, s0, s1, sem):
    wid = lax.axis_index("s") * 2 + lax.axis_index("c")
    t0 = wid * TOK_W
    pltpu.sync_copy(pos_hbm.at[0, pl.ds(t0, TOK_W)], p0)
    pltpu.sync_copy(pos_hbm.at[1, pl.ds(t0, TOK_W)], p1)
    pltpu.sync_copy(sc_hbm.at[0, pl.ds(t0, TOK_W)], s0)
    pltpu.sync_copy(sc_hbm.at[1, pl.ds(t0, TOK_W)], s1)
    cg0 = pltpu.async_copy(ys_hbm.at[p0], y0, sem)
    cg1 = pltpu.async_copy(ys_hbm.at[p1], y1, sem)
    cg0.wait()
    cg1.wait()

    @plsc.parallel_loop(0, TOK_W, unroll=2)
    def _(r):
        ridx = jnp.zeros((16,), jnp.int32) + r
        w0 = plsc.load_gather(s0, [ridx])
        w1 = plsc.load_gather(s1, [ridx])
        for c in range(D // 16):
            sl = pl.ds(16 * c, 16)
            y0[r, sl] = y0[r, sl] * w0 + y1[r, sl] * w1
    pltpu.sync_copy(y0, out_hbm.at[pl.ds(t0, TOK_W)])


def _combine(ys, pos, sc):
    mesh = plsc.VectorSubcoreMesh(core_axis_name="c", subcore_axis_name="s")
    f = pl.kernel(
        _combine_body,
        out_type=jax.ShapeDtypeStruct((N, D), jnp.float32),
        mesh=mesh,
        scratch_types=[
            pltpu.VMEM((TOK_W, D), jnp.float32),
            pltpu.VMEM((TOK_W, D), jnp.float32),
            pltpu.VMEM((TOK_W,), jnp.int32),
            pltpu.VMEM((TOK_W,), jnp.int32),
            pltpu.VMEM((TOK_W,), jnp.float32),
            pltpu.VMEM((TOK_W,), jnp.float32),
            pltpu.SemaphoreType.DMA,
        ],
        compiler_params=pltpu.CompilerParams(needs_layout_passes=False),
    )
    return f(ys, pos, sc)


def kernel(x, Wr, br, W1, b1, W2, b2):
    x2d = x.reshape(N, D)
    pos, sc, te8, trb8, nt8 = _router(x2d, Wr, br.reshape(1, E))
    te = te8[0, :G]
    trb = trb8[0, :G]
    nt = nt8[0, :1]
    xg = _dispatch(x2d, pos)
    ys = _gmm(te, trb, nt, xg, W1, b1, W2, b2)
    out = _combine(ys, pos, sc)
    return out.reshape(1, N, D)


# speedup vs baseline: 1.2013x; 1.0031x over previous
import jax
import jax.numpy as jnp
from jax import lax
from jax.experimental import pallas as pl
from jax.experimental.pallas import tpu as pltpu
from jax.experimental.pallas import tpu_sc as plsc

N = 2048
D = 768
H = 768
E = 64
T = 128
G = 95
NP = 12288
NW = 32
TOK_W = N // NW
NEG_INF = float("-inf")


def _router_body(x_ref, wr_ref, br_ref, pos_ref, sc_ref, te_ref,
                 trb_ref, nt_ref):
    xf = x_ref[...]
    logits = jnp.dot(xf, wr_ref[...], preferred_element_type=jnp.float32)
    logits = logits + br_ref[...]
    col = lax.broadcasted_iota(jnp.int32, (N, E), 1)

    m0 = jnp.max(logits, axis=1, keepdims=True)
    a0 = jnp.min(jnp.where(logits == m0, col, E), axis=1)
    oh0 = (col == a0[:, None]).astype(jnp.float32)
    neg = jnp.where(col == a0[:, None], NEG_INF, logits)
    m1 = jnp.max(neg, axis=1, keepdims=True)
    a1 = jnp.min(jnp.where(neg == m1, col, E), axis=1)
    oh1 = (col == a1[:, None]).astype(jnp.float32)

    t = jnp.exp(m1 - m0)
    w0 = 1.0 / (1.0 + t)
    w1 = t * w0

    cnt = jnp.sum(oh0, axis=0, keepdims=True) + jnp.sum(oh1, axis=0, keepdims=True)
    ecol = lax.broadcasted_iota(jnp.int32, (E, E), 1)
    erow = lax.broadcasted_iota(jnp.int32, (E, E), 0)
    ls_strict = (ecol < erow).astype(jnp.float32)
    ls_incl = (ecol <= erow).astype(jnp.float32)
    ntiles = jnp.floor((cnt + (T - 1)) * (1.0 / T))
    pc = ntiles * T
    off = jnp.dot(ls_strict, pc.reshape(E, 1),
                  preferred_element_type=jnp.float32).reshape(1, E)
    c_incl = jnp.dot(ls_incl, ntiles.reshape(E, 1),
                     preferred_element_type=jnp.float32).reshape(1, E)
    tse = c_incl - ntiles
    total_tiles = jnp.max(c_incl)

    gg = lax.broadcasted_iota(jnp.int32, (128, E), 0).astype(jnp.float32)
    gclamp = jnp.minimum(gg, total_tiles - 1.0)
    e_g = jnp.sum((c_incl <= gclamp).astype(jnp.float32), axis=1, keepdims=True)
    gcol = lax.broadcasted_iota(jnp.int32, (128, E), 1).astype(jnp.float32)
    ohg = (gcol == e_g).astype(jnp.float32)
    off_g = jnp.sum(ohg * off, axis=1, keepdims=True)
    tse_g = jnp.sum(ohg * tse, axis=1, keepdims=True)
    rb_g = off_g * (1.0 / T) + (gclamp[:, 0:1] - tse_g)
    te_ref[...] = jnp.broadcast_to(e_g.reshape(1, 128).astype(jnp.int32), (8, 128))
    trb_ref[...] = jnp.broadcast_to(rb_g.reshape(1, 128).astype(jnp.int32), (8, 128))
    nt_ref[...] = jnp.broadcast_to(total_tiles.astype(jnp.int32), (8, 128))

    nrow = lax.broadcasted_iota(jnp.int32, (N, N), 0)
    ncol = lax.broadcasted_iota(jnp.int32, (N, N), 1)
    ln_strict = (ncol < nrow).astype(jnp.bfloat16)
    oh01 = jnp.concatenate([oh0, oh1], axis=1).astype(jnp.bfloat16)
    r01 = jnp.dot(ln_strict, oh01, preferred_element_type=jnp.float32)
    tot0 = jnp.sum(oh0, axis=0, keepdims=True)

    rank0 = jnp.sum(r01[:, :E] * oh0, axis=1)
    rank1 = jnp.sum((r01[:, E:] + tot0) * oh1, axis=1)
    base0 = jnp.sum(oh0 * off, axis=1)
    base1 = jnp.sum(oh1 * off, axis=1)
    pos0 = (base0 + rank0).astype(jnp.int32)
    pos1 = (base1 + rank1).astype(jnp.int32)

    rowsel = lax.broadcasted_iota(jnp.int32, (8, N), 0)
    pos_ref[...] = jnp.where(rowsel == 0, pos0[None, :],
                             jnp.where(rowsel == 1, pos1[None, :], 0))
    sc_ref[...] = jnp.where(rowsel == 0, w0[:, 0][None, :],
                            jnp.where(rowsel == 1, w1[:, 0][None, :], 0.0))


def _router(x2d, Wr, br2d):
    return pl.pallas_call(
        _router_body,
        out_shape=(
            jax.ShapeDtypeStruct((8, N), jnp.int32),
            jax.ShapeDtypeStruct((8, N), jnp.float32),
            jax.ShapeDtypeStruct((8, 128), jnp.int32),
            jax.ShapeDtypeStruct((8, 128), jnp.int32),
            jax.ShapeDtypeStruct((8, 128), jnp.int32),
        ),
    )(x2d, Wr, br2d)


def _gmm_body(te_ref, trb_ref, nt_ref, xg_ref, w1_ref, b1_ref, w2_ref, b2_ref,
              y_ref):
    del te_ref, trb_ref

    @pl.when(pl.program_id(0) < nt_ref[0])
    def _():
        xb = xg_ref[...].astype(jnp.bfloat16)
        h = jnp.dot(xb, w1_ref[0].astype(jnp.bfloat16),
                    preferred_element_type=jnp.float32) + b1_ref[0]
        h = jnp.maximum(h, 0.0).astype(jnp.bfloat16)
        y = jnp.dot(h, w2_ref[0].astype(jnp.bfloat16),
                    preferred_element_type=jnp.float32) + b2_ref[0]
        y_ref[...] = y


def _gmm(te, trb, nt, xg, W1, b1, W2, b2):
    grid_spec = pltpu.PrefetchScalarGridSpec(
        num_scalar_prefetch=3,
        grid=(G,),
        in_specs=[
            pl.BlockSpec((T, D), lambda g, te, trb, nt: (trb[g], 0)),
            pl.BlockSpec((1, D, H), lambda g, te, trb, nt: (te[g], 0, 0)),
            pl.BlockSpec((1, 1, H), lambda g, te, trb, nt: (te[g], 0, 0)),
            pl.BlockSpec((1, H, D), lambda g, te, trb, nt: (te[g], 0, 0)),
            pl.BlockSpec((1, 1, D), lambda g, te, trb, nt: (te[g], 0, 0)),
        ],
        out_specs=pl.BlockSpec((T, D), lambda g, te, trb, nt: (trb[g], 0)),
    )
    return pl.pallas_call(
        _gmm_body,
        grid_spec=grid_spec,
        out_shape=jax.ShapeDtypeStruct((NP, D), jnp.float32),
        compiler_params=pltpu.CompilerParams(
            dimension_semantics=("arbitrary",)),
    )(te, trb, nt, xg, W1, b1.reshape(E, 1, H), W2, b2.reshape(E, 1, D))


def _dispatch_body(x_hbm, pos_hbm, xg_hbm, xloc, p0, p1, sem):
    wid = lax.axis_index("s") * 2 + lax.axis_index("c")
    t0 = wid * TOK_W
    pltpu.sync_copy(x_hbm.at[pl.ds(t0, TOK_W)], xloc)
    pltpu.sync_copy(pos_hbm.at[0, pl.ds(t0, TOK_W)], p0)
    pltpu.sync_copy(pos_hbm.at[1, pl.ds(t0, TOK_W)], p1)
    cp0 = pltpu.async_copy(xloc, xg_hbm.at[p0], sem)
    cp1 = pltpu.async_copy(xloc, xg_hbm.at[p1], sem)
    cp0.wait()
    cp1.wait()


def _dispatch(x2d, pos):
    mesh = plsc.VectorSubcoreMesh(core_axis_name="c", subcore_axis_name="s")
    f = pl.kernel(
        _dispatch_body,
        out_type=jax.ShapeDtypeStruct((NP, D), jnp.float32),
        mesh=mesh,
        scratch_types=[
            pltpu.VMEM((TOK_W, D), jnp.float32),
            pltpu.VMEM((TOK_W,), jnp.int32),
            pltpu.VMEM((TOK_W,), jnp.int32),
            pltpu.SemaphoreType.DMA,
        ],
    )
    return f(x2d, pos)


def _combine_body(ys_hbm, pos_hbm, sc_hbm, out_hbm, y0, y1, p0, p1, s0, s1, sem):
    wid = lax.axis_index("s") * 2 + lax.axis_index("c")
    t0 = wid * TOK_W
    pltpu.sync_copy(pos_hbm.at[0, pl.ds(t0, TOK_W)], p0)
    pltpu.sync_copy(pos_hbm.at[1, pl.ds(t0, TOK_W)], p1)
    pltpu.sync_copy(sc_hbm.at[0, pl.ds(t0, TOK_W)], s0)
    pltpu.sync_copy(sc_hbm.at[1, pl.ds(t0, TOK_W)], s1)
    cg0 = pltpu.async_copy(ys_hbm.at[p0], y0, sem)
    cg1 = pltpu.async_copy(ys_hbm.at[p1], y1, sem)
    cg0.wait()
    cg1.wait()

    @plsc.parallel_loop(0, TOK_W, unroll=2)
    def _(r):
        ridx = jnp.zeros((16,), jnp.int32) + r
        w0 = plsc.load_gather(s0, [ridx])
        w1 = plsc.load_gather(s1, [ridx])
        for c in range(D // 16):
            sl = pl.ds(16 * c, 16)
            y0[r, sl] = y0[r, sl] * w0 + y1[r, sl] * w1
    pltpu.sync_copy(y0, out_hbm.at[pl.ds(t0, TOK_W)])


def _combine(ys, pos, sc):
    mesh = plsc.VectorSubcoreMesh(core_axis_name="c", subcore_axis_name="s")
    f = pl.kernel(
        _combine_body,
        out_type=jax.ShapeDtypeStruct((N, D), jnp.float32),
        mesh=mesh,
        scratch_types=[
            pltpu.VMEM((TOK_W, D), jnp.float32),
            pltpu.VMEM((TOK_W, D), jnp.float32),
            pltpu.VMEM((TOK_W,), jnp.int32),
            pltpu.VMEM((TOK_W,), jnp.int32),
            pltpu.VMEM((TOK_W,), jnp.float32),
            pltpu.VMEM((TOK_W,), jnp.float32),
            pltpu.SemaphoreType.DMA,
        ],
        compiler_params=pltpu.CompilerParams(needs_layout_passes=False),
    )
    return f(ys, pos, sc)


def kernel(x, Wr, br, W1, b1, W2, b2):
    x2d = x.reshape(N, D)
    pos, sc, te8, trb8, nt8 = _router(x2d, Wr, br.reshape(1, E))
    te = te8[0, :G]
    trb = trb8[0, :G]
    nt = nt8[0, :1]
    xg = _dispatch(x2d, pos)
    ys = _gmm(te, trb, nt, xg, W1, b1, W2, b2)
    out = _combine(ys, pos, sc)
    return out.reshape(1, N, D)
